# baseline (speedup 1.0000x reference)
import functools

import jax
import jax.numpy as jnp
from jax import lax
from jax.experimental import pallas as pl
from jax.experimental.pallas import tpu as pltpu
from jax.experimental.pallas import tpu_sc as plsc

V = 100000
D = 64
WIN = 20
DH = 128
B = 1024

TV = 2048
NVB = (V + TV - 1) // TV


def _sc_gather(emb, idx_flat):
    info = plsc.get_sparse_core_info()
    nw = info.num_cores * info.num_subcores
    n = idx_flat.shape[0]
    b_per_w = n // nw
    mesh = plsc.VectorSubcoreMesh(core_axis_name="c", subcore_axis_name="s")

    @functools.partial(
        pl.kernel,
        mesh=mesh,
        out_type=jax.ShapeDtypeStruct((n, D), jnp.float32),
        compiler_params=pltpu.CompilerParams(use_tc_tiling_on_sc=False),
        scratch_types=[
            pltpu.VMEM((b_per_w,), jnp.int32),
            pltpu.VMEM((b_per_w, D), jnp.float32),
            pltpu.SemaphoreType.DMA,
        ],
    )
    def gather_kernel(table_hbm, idx_hbm, out_hbm, idx_v, rows_v, sem):
        wid = lax.axis_index("s") * info.num_cores + lax.axis_index("c")
        base = wid * b_per_w
        pltpu.sync_copy(idx_hbm.at[pl.ds(base, b_per_w)], idx_v)
        pltpu.async_copy(table_hbm.at[idx_v], rows_v, sem).wait()
        pltpu.sync_copy(rows_v, out_hbm.at[pl.ds(base, b_per_w)])

    return gather_kernel(emb, idx_flat)


def _mlp_body(e_ref, w1_ref, b1_ref, w2_ref, out_ref, h_ref):
    @pl.when(pl.program_id(0) == 0)
    def _():
        h = lax.dot_general(
            e_ref[...], w1_ref[...], (((1,), (1,)), ((), ())),
            preferred_element_type=jnp.float32)
        h_ref[...] = jnp.maximum(h + b1_ref[...], 0.0)

    out_ref[...] = lax.dot_general(
        h_ref[...], w2_ref[...], (((1,), (1,)), ((), ())),
        preferred_element_type=jnp.float32)


def _mlp(e_flat, W1, b1, W2):
    return pl.pallas_call(
        _mlp_body,
        grid=(NVB,),
        in_specs=[
            pl.BlockSpec((B, WIN * D), lambda j: (0, 0)),
            pl.BlockSpec((DH, WIN * D), lambda j: (0, 0)),
            pl.BlockSpec((1, DH), lambda j: (0, 0)),
            pl.BlockSpec((TV, DH), lambda j: (j, 0)),
        ],
        out_specs=pl.BlockSpec((B, TV), lambda j: (0, j)),
        out_shape=jax.ShapeDtypeStruct((B, V), jnp.float32),
        scratch_shapes=[pltpu.VMEM((B, DH), jnp.float32)],
        compiler_params=pltpu.CompilerParams(
            dimension_semantics=("arbitrary",)),
    )(e_flat, W1, b1, W2)


def kernel(x, emb, W1, b1, W2):
    idx_flat = x.reshape(-1).astype(jnp.int32)
    rows = _sc_gather(emb, idx_flat)
    e_flat = rows.reshape(B, WIN * D)
    b1_2d = b1.reshape(1, DH)
    return _mlp(e_flat, W1, b1_2d, W2)


# speedup vs baseline: 2.4855x; 2.4855x over previous
import functools

import jax
import jax.numpy as jnp
from jax import lax
from jax.experimental import pallas as pl
from jax.experimental.pallas import tpu as pltpu
from jax.experimental.pallas import tpu_sc as plsc

V = 100000
D = 64
WIN = 20
DH = 128
B = 1024

TV = 2048
NVB = (V + TV - 1) // TV


def _sc_gather(emb, idx_flat):
    info = plsc.get_sparse_core_info()
    nw = info.num_cores * info.num_subcores
    n = idx_flat.shape[0]
    b_per_w = n // nw
    mesh = plsc.VectorSubcoreMesh(core_axis_name="c", subcore_axis_name="s")

    @functools.partial(
        pl.kernel,
        mesh=mesh,
        out_type=jax.ShapeDtypeStruct((n, D), jnp.float32),
        compiler_params=pltpu.CompilerParams(use_tc_tiling_on_sc=False),
        scratch_types=[
            pltpu.VMEM((b_per_w,), jnp.int32),
            pltpu.VMEM((b_per_w, D), jnp.float32),
            pltpu.SemaphoreType.DMA,
        ],
    )
    def gather_kernel(table_hbm, idx_hbm, out_hbm, idx_v, rows_v, sem):
        wid = lax.axis_index("s") * info.num_cores + lax.axis_index("c")
        base = wid * b_per_w
        pltpu.sync_copy(idx_hbm.at[pl.ds(base, b_per_w)], idx_v)
        pltpu.async_copy(table_hbm.at[idx_v], rows_v, sem).wait()
        pltpu.sync_copy(rows_v, out_hbm.at[pl.ds(base, b_per_w)])

    return gather_kernel(emb, idx_flat)


def _mlp_body(e_ref, w1_ref, b1_ref, w2_ref, out_ref, h_ref):
    @pl.when(pl.program_id(0) == 0)
    def _():
        h = lax.dot_general(
            e_ref[...], w1_ref[...], (((1,), (1,)), ((), ())),
            preferred_element_type=jnp.float32)
        h_ref[...] = jnp.maximum(h + b1_ref[...], 0.0)

    out_ref[...] = lax.dot_general(
        w2_ref[...], h_ref[...], (((1,), (1,)), ((), ())),
        preferred_element_type=jnp.float32)


def _mlp(e_flat, W1, b1, W2):
    return pl.pallas_call(
        _mlp_body,
        grid=(NVB,),
        in_specs=[
            pl.BlockSpec((B, WIN * D), lambda j: (0, 0)),
            pl.BlockSpec((DH, WIN * D), lambda j: (0, 0)),
            pl.BlockSpec((1, DH), lambda j: (0, 0)),
            pl.BlockSpec((TV, DH), lambda j: (j, 0)),
        ],
        out_specs=pl.BlockSpec((TV, B), lambda j: (j, 0)),
        out_shape=jax.ShapeDtypeStruct((V, B), jnp.float32),
        scratch_shapes=[pltpu.VMEM((B, DH), jnp.float32)],
        compiler_params=pltpu.CompilerParams(
            dimension_semantics=("arbitrary",)),
    )(e_flat, W1, b1, W2)


def kernel(x, emb, W1, b1, W2):
    idx_flat = x.reshape(-1).astype(jnp.int32)
    rows = _sc_gather(emb, idx_flat)
    e_flat = rows.reshape(B, WIN * D)
    b1_2d = b1.reshape(1, DH)
    return _mlp(e_flat, W1, b1_2d, W2).T


# speedup vs baseline: 2.5175x; 1.0129x over previous
import functools

import jax
import jax.numpy as jnp
from jax import lax
from jax.experimental import pallas as pl
from jax.experimental.pallas import tpu as pltpu
from jax.experimental.pallas import tpu_sc as plsc

V = 100000
D = 64
WIN = 20
DH = 128
B = 1024

TV = 4096
NVB = (V + TV - 1) // TV


def _sc_gather(emb, idx_flat):
    info = plsc.get_sparse_core_info()
    nw = info.num_cores * info.num_subcores
    n = idx_flat.shape[0]
    b_per_w = n // nw
    mesh = plsc.VectorSubcoreMesh(core_axis_name="c", subcore_axis_name="s")

    @functools.partial(
        pl.kernel,
        mesh=mesh,
        out_type=jax.ShapeDtypeStruct((n, D), jnp.float32),
        compiler_params=pltpu.CompilerParams(use_tc_tiling_on_sc=False),
        scratch_types=[
            pltpu.VMEM((b_per_w,), jnp.int32),
            pltpu.VMEM((b_per_w, D), jnp.float32),
            pltpu.SemaphoreType.DMA,
        ],
    )
    def gather_kernel(table_hbm, idx_hbm, out_hbm, idx_v, rows_v, sem):
        wid = lax.axis_index("s") * info.num_cores + lax.axis_index("c")
        base = wid * b_per_w
        pltpu.sync_copy(idx_hbm.at[pl.ds(base, b_per_w)], idx_v)
        pltpu.async_copy(table_hbm.at[idx_v], rows_v, sem).wait()
        pltpu.sync_copy(rows_v, out_hbm.at[pl.ds(base, b_per_w)])

    return gather_kernel(emb, idx_flat)


def _mlp_body(e_ref, w1_ref, b1_ref, w2_ref, out_ref, h_ref):
    @pl.when(pl.program_id(0) == 0)
    def _():
        h = lax.dot_general(
            e_ref[...], w1_ref[...], (((1,), (1,)), ((), ())),
            preferred_element_type=jnp.float32)
        h_ref[...] = jnp.maximum(h + b1_ref[...], 0.0)

    out_ref[...] = lax.dot_general(
        w2_ref[...], h_ref[...], (((1,), (1,)), ((), ())),
        preferred_element_type=jnp.float32)


def _mlp(e_flat, W1, b1, W2):
    return pl.pallas_call(
        _mlp_body,
        grid=(NVB,),
        in_specs=[
            pl.BlockSpec((B, WIN * D), lambda j: (0, 0)),
            pl.BlockSpec((DH, WIN * D), lambda j: (0, 0)),
            pl.BlockSpec((1, DH), lambda j: (0, 0)),
            pl.BlockSpec((TV, DH), lambda j: (j, 0)),
        ],
        out_specs=pl.BlockSpec((TV, B), lambda j: (j, 0)),
        out_shape=jax.ShapeDtypeStruct((V, B), jnp.float32),
        scratch_shapes=[pltpu.VMEM((B, DH), jnp.float32)],
        compiler_params=pltpu.CompilerParams(
            dimension_semantics=("arbitrary",)),
    )(e_flat, W1, b1, W2)


def kernel(x, emb, W1, b1, W2):
    idx_flat = x.reshape(-1).astype(jnp.int32)
    rows = _sc_gather(emb, idx_flat)
    e_flat = rows.reshape(B, WIN * D)
    b1_2d = b1.reshape(1, DH)
    return _mlp(e_flat, W1, b1_2d, W2).T
